# Initial kernel scaffold; baseline (speedup 1.0000x reference)
import jax
import jax.numpy as jnp
from jax.experimental import pallas as pl


def kernel(x_inp, edge_index, edge_attr, node2graph, params):
    raise NotImplementedError("write your pallas kernel here")


# speedup vs baseline: 2.7338x; 2.7338x over previous
import functools

import jax
import jax.numpy as jnp
from jax import lax
from jax.experimental import pallas as pl
from jax.experimental.pallas import tpu as pltpu
from jax.experimental.pallas import tpu_sc as plsc

N = 10000
NE = 320000
FV = 128
FE = 16
H = 128
GV = 128
G = 64

BN = 1000
NB = N // BN
BE = 2000
NEB = NE // BE

_F32 = jnp.float32


def _dot(a, b):
    return jnp.dot(a, b, preferred_element_type=_F32)


def _onehot(n2g_row):
    iota = lax.broadcasted_iota(jnp.int32, (1, G), 1)
    return (n2g_row[:, None] == iota).astype(_F32)


def _emb_body(x_ref, w_ref, b_ref, o_ref):
    o_ref[...] = jax.nn.silu(_dot(x_ref[...], w_ref[...]) + b_ref[...])


def _node_emb(x_inp, w, b):
    return pl.pallas_call(
        _emb_body,
        grid=(NB,),
        in_specs=[pl.BlockSpec((BN, FV), lambda i: (i, 0)),
                  pl.BlockSpec((FV, H), lambda i: (0, 0)),
                  pl.BlockSpec((1, H), lambda i: (0, 0))],
        out_specs=pl.BlockSpec((BN, H), lambda i: (i, 0)),
        out_shape=jax.ShapeDtypeStruct((N, H), _F32),
    )(x_inp, w, b)


def _edge_mlp_body(ea_ref, we_ref, be_ref, w0_ref, b0_ref, w1_ref, b1_ref,
                   e0_ref, e1_ref):
    ea = jax.nn.silu(_dot(ea_ref[...], we_ref[...]) + be_ref[...])
    e0_ref[...] = jax.nn.silu(_dot(ea, w0_ref[...]) + b0_ref[...])
    e1_ref[...] = jax.nn.silu(_dot(ea, w1_ref[...]) + b1_ref[...])


def _edge_mlp(edge_attr, we, be, w0, b0, w1, b1):
    return pl.pallas_call(
        _edge_mlp_body,
        grid=(NEB,),
        in_specs=[pl.BlockSpec((BE, FE), lambda i: (i, 0)),
                  pl.BlockSpec((FE, H), lambda i: (0, 0)),
                  pl.BlockSpec((1, H), lambda i: (0, 0)),
                  pl.BlockSpec((H, H), lambda i: (0, 0)),
                  pl.BlockSpec((1, H), lambda i: (0, 0)),
                  pl.BlockSpec((H, H), lambda i: (0, 0)),
                  pl.BlockSpec((1, H), lambda i: (0, 0))],
        out_specs=[pl.BlockSpec((BE, H), lambda i: (i, 0)),
                   pl.BlockSpec((BE, H), lambda i: (i, 0))],
        out_shape=[jax.ShapeDtypeStruct((NE, H), _F32),
                   jax.ShapeDtypeStruct((NE, H), _F32)],
    )(edge_attr, we, be, w0, b0, w1, b1)


_NC = 2
_NS = 16
_NW = _NC * _NS
_EPW = NE // _NW
_CH = 80
_NCH = _EPW // _CH
_RPT = 632
_NP = _NS * _RPT


def _sc_conv_body(x_hbm, e_hbm, src_hbm, dst_hbm, zero_hbm, out_hbm,
                  src_v, dst_v, e_v, xg_v, acc, sem):
    c = lax.axis_index("c")
    s = lax.axis_index("s")
    wid = s * _NC + c
    r0 = s * _RPT
    pltpu.sync_copy(zero_hbm.at[pl.ds(r0, _RPT)], acc.at[pl.ds(r0, _RPT)])
    plsc.subcore_barrier()

    base = wid * _EPW

    def chunk(i, carry):
        off = base + i * _CH
        pltpu.sync_copy(src_hbm.at[pl.ds(off, _CH)], src_v)
        pltpu.sync_copy(dst_hbm.at[pl.ds(off, _CH)], dst_v)
        pltpu.sync_copy(e_hbm.at[pl.ds(off, _CH)], e_v)
        pltpu.async_copy(x_hbm.at[src_v], xg_v, sem).wait()

        def row(r, cc):
            for k in range(H // 16):
                sl = pl.ds(k * 16, 16)
                e_v[r, sl] = jnp.maximum(e_v[r, sl] + xg_v[r, sl], 0.0)
            return cc

        lax.fori_loop(0, _CH, row, 0)
        pltpu.sync_copy(e_v, acc.at[dst_v], add=True)
        return carry

    lax.fori_loop(0, _NCH, chunk, 0)
    plsc.subcore_barrier()
    pltpu.sync_copy(acc.at[pl.ds(r0, _RPT)], out_hbm.at[c, pl.ds(r0, _RPT)])


def _sc_conv(x, e, src, dst, zeros):
    f = pl.kernel(
        _sc_conv_body,
        out_type=jax.ShapeDtypeStruct((_NC, _NP, H), _F32),
        mesh=plsc.VectorSubcoreMesh(core_axis_name="c", subcore_axis_name="s"),
        scratch_types=[
            pltpu.VMEM((_CH,), jnp.int32),
            pltpu.VMEM((_CH,), jnp.int32),
            pltpu.VMEM((_CH, H), _F32),
            pltpu.VMEM((_CH, H), _F32),
            pltpu.VMEM_SHARED((_NP, H), _F32),
            pltpu.SemaphoreType.DMA,
        ],
    )
    return f(x, e, src, dst, zeros)


def _mlp_stats_body(scale_ref, parts_ref, x_ref, w1_ref, b1_ref, w2_ref,
                    b2_ref, n2g_ref, h_ref, stats_ref):
    i = pl.program_id(0)
    agg = parts_ref[0] + parts_ref[1]
    h = agg + scale_ref[0, 0] * x_ref[...]
    h = jax.nn.silu(_dot(h, w1_ref[...]) + b1_ref[...])
    h = jax.nn.silu(_dot(h, w2_ref[...]) + b2_ref[...])
    h_ref[...] = h
    oh = _onehot(n2g_ref[0, 0, :])
    dn = (((0,), (0,)), ((), ()))
    s1 = jnp.sum(lax.dot_general(oh, h, dn, preferred_element_type=_F32), 1)
    s2 = jnp.sum(lax.dot_general(oh, h * h, dn, preferred_element_type=_F32), 1)
    s0 = jnp.sum(oh, 0)
    blk = jnp.concatenate(
        [s0[None], s1[None], s2[None], jnp.zeros((5, G), _F32)], 0)

    @pl.when(i == 0)
    def _():
        stats_ref[...] = jnp.zeros_like(stats_ref)

    stats_ref[...] += blk


def _node_mlp_stats(scale, parts, x, w1, b1, w2, b2, n2g3):
    return pl.pallas_call(
        _mlp_stats_body,
        grid=(NB,),
        in_specs=[pl.BlockSpec((1, 1), lambda i: (0, 0)),
                  pl.BlockSpec((_NC, BN, H), lambda i: (0, i, 0)),
                  pl.BlockSpec((BN, H), lambda i: (i, 0)),
                  pl.BlockSpec((H, H), lambda i: (0, 0)),
                  pl.BlockSpec((1, H), lambda i: (0, 0)),
                  pl.BlockSpec((H, H), lambda i: (0, 0)),
                  pl.BlockSpec((1, H), lambda i: (0, 0)),
                  pl.BlockSpec((1, 1, BN), lambda i: (i, 0, 0))],
        out_specs=[pl.BlockSpec((BN, H), lambda i: (i, 0)),
                   pl.BlockSpec((8, G), lambda i: (0, 0))],
        out_shape=[jax.ShapeDtypeStruct((N, H), _F32),
                   jax.ShapeDtypeStruct((8, G), _F32)],
    )(scale, parts, x, w1, b1, w2, b2, n2g3)


def _norm_body(h_ref, x_ref, stats_ref, n2g_ref, w_ref, b_ref, o_ref):
    s0 = stats_ref[0, :]
    s1 = stats_ref[1, :]
    s2 = stats_ref[2, :]
    norm = jnp.maximum(s0, 1.0) * jnp.float32(H)
    mean = s1 / norm
    k = s0 * jnp.float32(H)
    var = (s2 - 2.0 * mean * s1 + mean * mean * k) / norm
    inv = lax.rsqrt(var + 1e-5)
    oh = _onehot(n2g_ref[0, 0, :])
    mean_n = jnp.sum(oh * mean[None, :], 1)
    inv_n = jnp.sum(oh * inv[None, :], 1)
    gn = (h_ref[...] - mean_n[:, None]) * inv_n[:, None] * w_ref[...] + b_ref[...]
    o_ref[...] = jnp.maximum((gn + x_ref[...]) * 0.5, 0.0)


def _apply_norm(h, x, stats, n2g3, w, b):
    return pl.pallas_call(
        _norm_body,
        grid=(NB,),
        in_specs=[pl.BlockSpec((BN, H), lambda i: (i, 0)),
                  pl.BlockSpec((BN, H), lambda i: (i, 0)),
                  pl.BlockSpec((8, G), lambda i: (0, 0)),
                  pl.BlockSpec((1, 1, BN), lambda i: (i, 0, 0)),
                  pl.BlockSpec((1, H), lambda i: (0, 0)),
                  pl.BlockSpec((1, H), lambda i: (0, 0))],
        out_specs=pl.BlockSpec((BN, H), lambda i: (i, 0)),
        out_shape=jax.ShapeDtypeStruct((N, H), _F32),
    )(h, x, stats, n2g3, w, b)


def _final_body(x2_ref, xin_ref, wf_ref, bf_ref, w1_ref, b1_ref, w2_ref,
                b2_ref, n2g_ref, xf_ref, z1_ref):
    i = pl.program_id(0)
    xf = (_dot(x2_ref[...], wf_ref[0:H, :])
          + _dot(xin_ref[...], wf_ref[H:H + FV, :]) + bf_ref[...])
    xf = jax.nn.silu(xf)
    xf_ref[...] = xf
    g1 = _dot(xf, w1_ref[...]) + b1_ref[...]
    g2 = _dot(xf, w2_ref[...]) + b2_ref[...]
    hh = g1 * jax.nn.sigmoid(g2)
    oh = _onehot(n2g_ref[0, 0, :])
    dn = (((0,), (0,)), ((), ()))
    blk = lax.dot_general(oh, hh, dn, preferred_element_type=_F32)

    @pl.when(i == 0)
    def _():
        z1_ref[...] = jnp.zeros_like(z1_ref)

    z1_ref[...] += blk


def _final_readout(x2, x_inp, wf, bf, w1, b1, w2, b2, n2g3):
    return pl.pallas_call(
        _final_body,
        grid=(NB,),
        in_specs=[pl.BlockSpec((BN, H), lambda i: (i, 0)),
                  pl.BlockSpec((BN, FV), lambda i: (i, 0)),
                  pl.BlockSpec((H + FV, H), lambda i: (0, 0)),
                  pl.BlockSpec((1, H), lambda i: (0, 0)),
                  pl.BlockSpec((H, GV), lambda i: (0, 0)),
                  pl.BlockSpec((1, GV), lambda i: (0, 0)),
                  pl.BlockSpec((H, GV), lambda i: (0, 0)),
                  pl.BlockSpec((1, GV), lambda i: (0, 0)),
                  pl.BlockSpec((1, 1, BN), lambda i: (i, 0, 0))],
        out_specs=[pl.BlockSpec((BN, H), lambda i: (i, 0)),
                   pl.BlockSpec((G, GV), lambda i: (0, 0))],
        out_shape=[jax.ShapeDtypeStruct((N, H), _F32),
                   jax.ShapeDtypeStruct((G, GV), _F32)],
    )(x2, x_inp, wf, bf, w1, b1, w2, b2, n2g3)


def _readout_mix_body(z1_ref, stats_ref, w3_ref, b3_ref, z_ref):
    c = jnp.maximum(stats_ref[0, :], 1.0)
    z1 = z1_ref[...]
    z2 = z1 / c[:, None]
    z = (_dot(z1, w3_ref[0:GV, :]) + _dot(z2, w3_ref[GV:2 * GV, :])
         + b3_ref[...])
    z_ref[...] = jax.nn.silu(z)


def _readout_mix(z1, stats, w3, b3):
    return pl.pallas_call(
        _readout_mix_body,
        in_specs=[pl.BlockSpec((G, GV), lambda: (0, 0)),
                  pl.BlockSpec((8, G), lambda: (0, 0)),
                  pl.BlockSpec((2 * GV, GV), lambda: (0, 0)),
                  pl.BlockSpec((1, GV), lambda: (0, 0))],
        out_specs=pl.BlockSpec((G, GV), lambda: (0, 0)),
        out_shape=jax.ShapeDtypeStruct((G, GV), _F32),
    )(z1, stats, w3, b3)


def kernel(x_inp, edge_index, edge_attr, node2graph, params):
    src = edge_index[0]
    dst = edge_index[1]
    n2g3 = node2graph.reshape(NB, 1, BN)
    zeros = jnp.zeros((_NP, H), _F32)

    p = params
    row = lambda v: v.reshape(1, -1)

    x = _node_emb(x_inp, p['node_emb']['W'], row(p['node_emb']['b']))

    b0c = p['blocks'][0]['conv2']
    b1c = p['blocks'][1]['conv2']
    e0, e1 = _edge_mlp(edge_attr, p['edge_emb']['W'], row(p['edge_emb']['b']),
                       b0c['We'], row(b0c['be']), b1c['We'], row(b1c['be']))

    stats = None
    for bp, e in ((p['blocks'][0], e0), (p['blocks'][1], e1)):
        cp = bp['conv2']
        parts = _sc_conv(x, e, src, dst, zeros)
        scale = (1.0 + cp['eps']).reshape(1, 1)
        h, stats = _node_mlp_stats(scale, parts, x, cp['W1'], row(cp['b1']),
                                   cp['W2'], row(cp['b2']), n2g3)
        x = _apply_norm(h, x, stats, n2g3, row(bp['norm2']['w']),
                        row(bp['norm2']['b']))

    r = p['readout']
    xf, z1 = _final_readout(x, x_inp, p['final']['W'], row(p['final']['b']),
                            r['W1'], row(r['b1']), r['W2'], row(r['b2']),
                            n2g3)
    z = _readout_mix(z1, stats, r['W3'], row(r['b3']))
    return (xf, z)


# speedup vs baseline: 3.7399x; 1.3680x over previous
import functools

import jax
import jax.numpy as jnp
from jax import lax
from jax.experimental import pallas as pl
from jax.experimental.pallas import tpu as pltpu
from jax.experimental.pallas import tpu_sc as plsc

N = 10000
NE = 320000
FV = 128
FE = 16
H = 128
GV = 128
G = 64

BN = 1000
NB = N // BN
BE = 2000
NEB = NE // BE

_F32 = jnp.float32


def _dot(a, b):
    return jnp.dot(a, b, preferred_element_type=_F32)


def _onehot(n2g_row):
    iota = lax.broadcasted_iota(jnp.int32, (1, G), 1)
    return (n2g_row[:, None] == iota).astype(_F32)


def _emb_body(x_ref, w_ref, b_ref, o_ref):
    o_ref[...] = jax.nn.silu(_dot(x_ref[...], w_ref[...]) + b_ref[...])


def _node_emb(x_inp, w, b):
    return pl.pallas_call(
        _emb_body,
        grid=(NB,),
        in_specs=[pl.BlockSpec((BN, FV), lambda i: (i, 0)),
                  pl.BlockSpec((FV, H), lambda i: (0, 0)),
                  pl.BlockSpec((1, H), lambda i: (0, 0))],
        out_specs=pl.BlockSpec((BN, H), lambda i: (i, 0)),
        out_shape=jax.ShapeDtypeStruct((N, H), _F32),
    )(x_inp, w, b)


def _edge_mlp_body(ea_ref, we_ref, be_ref, w0_ref, b0_ref, w1_ref, b1_ref,
                   e0_ref, e1_ref):
    ea = jax.nn.silu(_dot(ea_ref[...], we_ref[...]) + be_ref[...])
    e0_ref[...] = jax.nn.silu(_dot(ea, w0_ref[...]) + b0_ref[...])
    e1_ref[...] = jax.nn.silu(_dot(ea, w1_ref[...]) + b1_ref[...])


def _edge_mlp(edge_attr, we, be, w0, b0, w1, b1):
    return pl.pallas_call(
        _edge_mlp_body,
        grid=(NEB,),
        in_specs=[pl.BlockSpec((BE, FE), lambda i: (i, 0)),
                  pl.BlockSpec((FE, H), lambda i: (0, 0)),
                  pl.BlockSpec((1, H), lambda i: (0, 0)),
                  pl.BlockSpec((H, H), lambda i: (0, 0)),
                  pl.BlockSpec((1, H), lambda i: (0, 0)),
                  pl.BlockSpec((H, H), lambda i: (0, 0)),
                  pl.BlockSpec((1, H), lambda i: (0, 0))],
        out_specs=[pl.BlockSpec((BE, H), lambda i: (i, 0)),
                   pl.BlockSpec((BE, H), lambda i: (i, 0))],
        out_shape=[jax.ShapeDtypeStruct((NE, H), _F32),
                   jax.ShapeDtypeStruct((NE, H), _F32)],
    )(edge_attr, we, be, w0, b0, w1, b1)


_NC = 2
_NS = 16
_NW = _NC * _NS
_EPW = NE // _NW
_CH = 40
_NCH = _EPW // _CH
_NG = _NCH // 2
_RPT = 632
_NP = _NS * _RPT


def _sc_conv_body(x_hbm, e_hbm, src_hbm, dst_hbm, zero_hbm, out_hbm,
                  src_v, dst_v, e_v, xg_v, acc, isem, gsem, ssem):
    c = lax.axis_index("c")
    s = lax.axis_index("s")
    wid = s * _NC + c
    r0 = s * _RPT
    pltpu.sync_copy(zero_hbm.at[pl.ds(r0, _RPT)], acc.at[pl.ds(r0, _RPT)])
    plsc.subcore_barrier()

    base = wid * _EPW

    def issue_loads(i, b):
        off = base + i * _CH
        pltpu.async_copy(src_hbm.at[pl.ds(off, _CH)], src_v.at[b], isem.at[b])
        pltpu.async_copy(dst_hbm.at[pl.ds(off, _CH)], dst_v.at[b], isem.at[b])
        pltpu.async_copy(e_hbm.at[pl.ds(off, _CH)], e_v.at[b], isem.at[b])

    def wait_loads(i, b):
        off = base + i * _CH
        pltpu.make_async_copy(src_hbm.at[pl.ds(off, _CH)], src_v.at[b],
                              isem.at[b]).wait()
        pltpu.make_async_copy(dst_hbm.at[pl.ds(off, _CH)], dst_v.at[b],
                              isem.at[b]).wait()
        pltpu.make_async_copy(e_hbm.at[pl.ds(off, _CH)], e_v.at[b],
                              isem.at[b]).wait()

    def wait_scatter(b):
        pltpu.make_async_copy(e_v.at[b], acc.at[dst_v.at[b]],
                              ssem.at[b]).wait()

    def compute(b):
        def row(r, cc):
            for k in range(H // 16):
                sl = pl.ds(k * 16, 16)
                e_v[b, r, sl] = jnp.maximum(
                    e_v[b, r, sl] + xg_v[b, r, sl], 0.0)
            return cc

        lax.fori_loop(0, _CH, row, 0)

    issue_loads(0, 0)

    def outer(g, carry):
        i0 = g * 2
        wait_loads(i0, 0)
        pltpu.async_copy(x_hbm.at[src_v.at[0]], xg_v.at[0], gsem.at[0])

        @pl.when(g > 0)
        def _():
            wait_scatter(1)

        issue_loads(i0 + 1, 1)
        pltpu.make_async_copy(x_hbm.at[src_v.at[0]], xg_v.at[0],
                              gsem.at[0]).wait()
        compute(0)
        pltpu.async_copy(e_v.at[0], acc.at[dst_v.at[0]], ssem.at[0], add=True)

        wait_loads(i0 + 1, 1)
        pltpu.async_copy(x_hbm.at[src_v.at[1]], xg_v.at[1], gsem.at[1])

        @pl.when(g < _NG - 1)
        def _():
            wait_scatter(0)
            issue_loads(i0 + 2, 0)

        pltpu.make_async_copy(x_hbm.at[src_v.at[1]], xg_v.at[1],
                              gsem.at[1]).wait()
        compute(1)
        pltpu.async_copy(e_v.at[1], acc.at[dst_v.at[1]], ssem.at[1], add=True)
        return carry

    lax.fori_loop(0, _NG, outer, 0)
    wait_scatter(0)
    wait_scatter(1)
    plsc.subcore_barrier()
    pltpu.sync_copy(acc.at[pl.ds(r0, _RPT)], out_hbm.at[c, pl.ds(r0, _RPT)])


def _sc_conv(x, e, src, dst, zeros):
    f = pl.kernel(
        _sc_conv_body,
        out_type=jax.ShapeDtypeStruct((_NC, _NP, H), _F32),
        mesh=plsc.VectorSubcoreMesh(core_axis_name="c", subcore_axis_name="s"),
        scratch_types=[
            pltpu.VMEM((2, _CH), jnp.int32),
            pltpu.VMEM((2, _CH), jnp.int32),
            pltpu.VMEM((2, _CH, H), _F32),
            pltpu.VMEM((2, _CH, H), _F32),
            pltpu.VMEM_SHARED((_NP, H), _F32),
            pltpu.SemaphoreType.DMA((2,)),
            pltpu.SemaphoreType.DMA((2,)),
            pltpu.SemaphoreType.DMA((2,)),
        ],
    )
    return f(x, e, src, dst, zeros)


def _mlp_stats_body(scale_ref, parts_ref, x_ref, w1_ref, b1_ref, w2_ref,
                    b2_ref, n2g_ref, h_ref, stats_ref):
    i = pl.program_id(0)
    agg = parts_ref[0] + parts_ref[1]
    h = agg + scale_ref[0, 0] * x_ref[...]
    h = jax.nn.silu(_dot(h, w1_ref[...]) + b1_ref[...])
    h = jax.nn.silu(_dot(h, w2_ref[...]) + b2_ref[...])
    h_ref[...] = h
    oh = _onehot(n2g_ref[0, 0, :])
    dn = (((0,), (0,)), ((), ()))
    s1 = jnp.sum(lax.dot_general(oh, h, dn, preferred_element_type=_F32), 1)
    s2 = jnp.sum(lax.dot_general(oh, h * h, dn, preferred_element_type=_F32), 1)
    s0 = jnp.sum(oh, 0)
    blk = jnp.concatenate(
        [s0[None], s1[None], s2[None], jnp.zeros((5, G), _F32)], 0)

    @pl.when(i == 0)
    def _():
        stats_ref[...] = jnp.zeros_like(stats_ref)

    stats_ref[...] += blk


def _node_mlp_stats(scale, parts, x, w1, b1, w2, b2, n2g3):
    return pl.pallas_call(
        _mlp_stats_body,
        grid=(NB,),
        in_specs=[pl.BlockSpec((1, 1), lambda i: (0, 0)),
                  pl.BlockSpec((_NC, BN, H), lambda i: (0, i, 0)),
                  pl.BlockSpec((BN, H), lambda i: (i, 0)),
                  pl.BlockSpec((H, H), lambda i: (0, 0)),
                  pl.BlockSpec((1, H), lambda i: (0, 0)),
                  pl.BlockSpec((H, H), lambda i: (0, 0)),
                  pl.BlockSpec((1, H), lambda i: (0, 0)),
                  pl.BlockSpec((1, 1, BN), lambda i: (i, 0, 0))],
        out_specs=[pl.BlockSpec((BN, H), lambda i: (i, 0)),
                   pl.BlockSpec((8, G), lambda i: (0, 0))],
        out_shape=[jax.ShapeDtypeStruct((N, H), _F32),
                   jax.ShapeDtypeStruct((8, G), _F32)],
    )(scale, parts, x, w1, b1, w2, b2, n2g3)


def _norm_body(h_ref, x_ref, stats_ref, n2g_ref, w_ref, b_ref, o_ref):
    s0 = stats_ref[0, :]
    s1 = stats_ref[1, :]
    s2 = stats_ref[2, :]
    norm = jnp.maximum(s0, 1.0) * jnp.float32(H)
    mean = s1 / norm
    k = s0 * jnp.float32(H)
    var = (s2 - 2.0 * mean * s1 + mean * mean * k) / norm
    inv = lax.rsqrt(var + 1e-5)
    oh = _onehot(n2g_ref[0, 0, :])
    mean_n = jnp.sum(oh * mean[None, :], 1)
    inv_n = jnp.sum(oh * inv[None, :], 1)
    gn = (h_ref[...] - mean_n[:, None]) * inv_n[:, None] * w_ref[...] + b_ref[...]
    o_ref[...] = jnp.maximum((gn + x_ref[...]) * 0.5, 0.0)


def _apply_norm(h, x, stats, n2g3, w, b):
    return pl.pallas_call(
        _norm_body,
        grid=(NB,),
        in_specs=[pl.BlockSpec((BN, H), lambda i: (i, 0)),
                  pl.BlockSpec((BN, H), lambda i: (i, 0)),
                  pl.BlockSpec((8, G), lambda i: (0, 0)),
                  pl.BlockSpec((1, 1, BN), lambda i: (i, 0, 0)),
                  pl.BlockSpec((1, H), lambda i: (0, 0)),
                  pl.BlockSpec((1, H), lambda i: (0, 0))],
        out_specs=pl.BlockSpec((BN, H), lambda i: (i, 0)),
        out_shape=jax.ShapeDtypeStruct((N, H), _F32),
    )(h, x, stats, n2g3, w, b)


def _final_body(x2_ref, xin_ref, wf_ref, bf_ref, w1_ref, b1_ref, w2_ref,
                b2_ref, n2g_ref, xf_ref, z1_ref):
    i = pl.program_id(0)
    xf = (_dot(x2_ref[...], wf_ref[0:H, :])
          + _dot(xin_ref[...], wf_ref[H:H + FV, :]) + bf_ref[...])
    xf = jax.nn.silu(xf)
    xf_ref[...] = xf
    g1 = _dot(xf, w1_ref[...]) + b1_ref[...]
    g2 = _dot(xf, w2_ref[...]) + b2_ref[...]
    hh = g1 * jax.nn.sigmoid(g2)
    oh = _onehot(n2g_ref[0, 0, :])
    dn = (((0,), (0,)), ((), ()))
    blk = lax.dot_general(oh, hh, dn, preferred_element_type=_F32)

    @pl.when(i == 0)
    def _():
        z1_ref[...] = jnp.zeros_like(z1_ref)

    z1_ref[...] += blk


def _final_readout(x2, x_inp, wf, bf, w1, b1, w2, b2, n2g3):
    return pl.pallas_call(
        _final_body,
        grid=(NB,),
        in_specs=[pl.BlockSpec((BN, H), lambda i: (i, 0)),
                  pl.BlockSpec((BN, FV), lambda i: (i, 0)),
                  pl.BlockSpec((H + FV, H), lambda i: (0, 0)),
                  pl.BlockSpec((1, H), lambda i: (0, 0)),
                  pl.BlockSpec((H, GV), lambda i: (0, 0)),
                  pl.BlockSpec((1, GV), lambda i: (0, 0)),
                  pl.BlockSpec((H, GV), lambda i: (0, 0)),
                  pl.BlockSpec((1, GV), lambda i: (0, 0)),
                  pl.BlockSpec((1, 1, BN), lambda i: (i, 0, 0))],
        out_specs=[pl.BlockSpec((BN, H), lambda i: (i, 0)),
                   pl.BlockSpec((G, GV), lambda i: (0, 0))],
        out_shape=[jax.ShapeDtypeStruct((N, H), _F32),
                   jax.ShapeDtypeStruct((G, GV), _F32)],
    )(x2, x_inp, wf, bf, w1, b1, w2, b2, n2g3)


def _readout_mix_body(z1_ref, stats_ref, w3_ref, b3_ref, z_ref):
    c = jnp.maximum(stats_ref[0, :], 1.0)
    z1 = z1_ref[...]
    z2 = z1 / c[:, None]
    z = (_dot(z1, w3_ref[0:GV, :]) + _dot(z2, w3_ref[GV:2 * GV, :])
         + b3_ref[...])
    z_ref[...] = jax.nn.silu(z)


def _readout_mix(z1, stats, w3, b3):
    return pl.pallas_call(
        _readout_mix_body,
        in_specs=[pl.BlockSpec((G, GV), lambda: (0, 0)),
                  pl.BlockSpec((8, G), lambda: (0, 0)),
                  pl.BlockSpec((2 * GV, GV), lambda: (0, 0)),
                  pl.BlockSpec((1, GV), lambda: (0, 0))],
        out_specs=pl.BlockSpec((G, GV), lambda: (0, 0)),
        out_shape=jax.ShapeDtypeStruct((G, GV), _F32),
    )(z1, stats, w3, b3)


def kernel(x_inp, edge_index, edge_attr, node2graph, params):
    src = edge_index[0]
    dst = edge_index[1]
    n2g3 = node2graph.reshape(NB, 1, BN)
    zeros = jnp.zeros((_NP, H), _F32)

    p = params
    row = lambda v: v.reshape(1, -1)

    x = _node_emb(x_inp, p['node_emb']['W'], row(p['node_emb']['b']))

    b0c = p['blocks'][0]['conv2']
    b1c = p['blocks'][1]['conv2']
    e0, e1 = _edge_mlp(edge_attr, p['edge_emb']['W'], row(p['edge_emb']['b']),
                       b0c['We'], row(b0c['be']), b1c['We'], row(b1c['be']))

    stats = None
    for bp, e in ((p['blocks'][0], e0), (p['blocks'][1], e1)):
        cp = bp['conv2']
        parts = _sc_conv(x, e, src, dst, zeros)
        scale = (1.0 + cp['eps']).reshape(1, 1)
        h, stats = _node_mlp_stats(scale, parts, x, cp['W1'], row(cp['b1']),
                                   cp['W2'], row(cp['b2']), n2g3)
        x = _apply_norm(h, x, stats, n2g3, row(bp['norm2']['w']),
                        row(bp['norm2']['b']))

    r = p['readout']
    xf, z1 = _final_readout(x, x_inp, p['final']['W'], row(p['final']['b']),
                            r['W1'], row(r['b1']), r['W2'], row(r['b2']),
                            n2g3)
    z = _readout_mix(z1, stats, r['W3'], row(r['b3']))
    return (xf, z)


# speedup vs baseline: 4.3748x; 1.1698x over previous
import functools

import jax
import jax.numpy as jnp
from jax import lax
from jax.experimental import pallas as pl
from jax.experimental.pallas import tpu as pltpu
from jax.experimental.pallas import tpu_sc as plsc

N = 10000
NE = 320000
FV = 128
FE = 16
H = 128
GV = 128
G = 64

BN = 1000
NB = N // BN
BE = 2000
NEB = NE // BE

_F32 = jnp.float32


def _dot(a, b):
    return jnp.dot(a, b, preferred_element_type=_F32)


def _onehot(n2g_row):
    iota = lax.broadcasted_iota(jnp.int32, (1, G), 1)
    return (n2g_row[:, None] == iota).astype(_F32)


def _emb_body(x_ref, w_ref, b_ref, o_ref):
    o_ref[...] = jax.nn.silu(_dot(x_ref[...], w_ref[...]) + b_ref[...])


def _node_emb(x_inp, w, b):
    return pl.pallas_call(
        _emb_body,
        grid=(NB,),
        in_specs=[pl.BlockSpec((BN, FV), lambda i: (i, 0)),
                  pl.BlockSpec((FV, H), lambda i: (0, 0)),
                  pl.BlockSpec((1, H), lambda i: (0, 0))],
        out_specs=pl.BlockSpec((BN, H), lambda i: (i, 0)),
        out_shape=jax.ShapeDtypeStruct((N, H), _F32),
    )(x_inp, w, b)


def _edge_mlp_body(ea_ref, we_ref, be_ref, wcat_ref, bcat_ref, e0_ref, e1_ref):
    ea = jax.nn.silu(_dot(ea_ref[...], we_ref[...]) + be_ref[...])
    z = jnp.dot(ea.astype(jnp.bfloat16), wcat_ref[...],
                preferred_element_type=_F32) + bcat_ref[...]
    z = jax.nn.silu(z)
    e0_ref[...] = z[:, 0:H]
    e1_ref[...] = z[:, H:2 * H]


def _edge_mlp(edge_attr, we, be, wcat, bcat):
    return pl.pallas_call(
        _edge_mlp_body,
        grid=(NEB,),
        in_specs=[pl.BlockSpec((BE, FE), lambda i: (i, 0)),
                  pl.BlockSpec((FE, H), lambda i: (0, 0)),
                  pl.BlockSpec((1, H), lambda i: (0, 0)),
                  pl.BlockSpec((H, 2 * H), lambda i: (0, 0)),
                  pl.BlockSpec((1, 2 * H), lambda i: (0, 0))],
        out_specs=[pl.BlockSpec((BE, H), lambda i: (i, 0)),
                   pl.BlockSpec((BE, H), lambda i: (i, 0))],
        out_shape=[jax.ShapeDtypeStruct((NE, H), _F32),
                   jax.ShapeDtypeStruct((NE, H), _F32)],
    )(edge_attr, we, be, wcat, bcat)


_NC = 2
_NS = 16
_NW = _NC * _NS
_EPW = NE // _NW
_CH = 80
_NCH = _EPW // _CH
_NG = _NCH // 2
_RPT = 632
_NP = _NS * _RPT


def _sc_conv_body(x_hbm, e_hbm, src_hbm, dst_hbm, zero_hbm, out_hbm,
                  src_v, dst_v, e_v, xg_v, acc, isem, gsem, ssem):
    c = lax.axis_index("c")
    s = lax.axis_index("s")
    wid = s * _NC + c
    r0 = s * _RPT
    pltpu.sync_copy(zero_hbm.at[pl.ds(r0, _RPT)], acc.at[pl.ds(r0, _RPT)])
    plsc.subcore_barrier()

    base = wid * _EPW

    def issue_loads(i, b):
        off = base + i * _CH
        pltpu.async_copy(src_hbm.at[pl.ds(off, _CH)], src_v.at[b], isem.at[b])
        pltpu.async_copy(dst_hbm.at[pl.ds(off, _CH)], dst_v.at[b], isem.at[b])
        pltpu.async_copy(e_hbm.at[pl.ds(off, _CH)], e_v.at[b], isem.at[b])

    def wait_loads(i, b):
        off = base + i * _CH
        pltpu.make_async_copy(src_hbm.at[pl.ds(off, _CH)], src_v.at[b],
                              isem.at[b]).wait()
        pltpu.make_async_copy(dst_hbm.at[pl.ds(off, _CH)], dst_v.at[b],
                              isem.at[b]).wait()
        pltpu.make_async_copy(e_hbm.at[pl.ds(off, _CH)], e_v.at[b],
                              isem.at[b]).wait()

    def issue_gather(b):
        pltpu.async_copy(x_hbm.at[src_v.at[b]], xg_v.at[b], gsem.at[b])

    def wait_gather(b):
        pltpu.make_async_copy(x_hbm.at[src_v.at[b]], xg_v.at[b],
                              gsem.at[b]).wait()

    def issue_scatter(b):
        pltpu.async_copy(e_v.at[b], acc.at[dst_v.at[b]], ssem.at[b],
                         add=True)

    def wait_scatter(b):
        pltpu.make_async_copy(e_v.at[b], acc.at[dst_v.at[b]],
                              ssem.at[b]).wait()

    def compute(b):
        def row(r, cc):
            for t in range(H // 16):
                sl = pl.ds(t * 16, 16)
                e_v[b, r, sl] = jnp.maximum(
                    e_v[b, r, sl] + xg_v[b, r, sl], 0.0)
            return cc

        lax.fori_loop(0, _CH, row, 0)

    issue_loads(0, 0)

    def outer(g, carry):
        i0 = g * 2
        wait_loads(i0, 0)
        issue_gather(0)

        @pl.when(g > 0)
        def _():
            wait_scatter(1)

        issue_loads(i0 + 1, 1)
        wait_gather(0)
        compute(0)
        issue_scatter(0)

        wait_loads(i0 + 1, 1)
        issue_gather(1)
        wait_scatter(0)
        issue_loads(i0 + 2, 0)
        wait_gather(1)
        compute(1)
        issue_scatter(1)
        return carry

    lax.fori_loop(0, _NG, outer, 0)
    wait_loads(_NCH - 1, 0)
    issue_gather(0)
    wait_scatter(1)
    wait_gather(0)
    compute(0)
    issue_scatter(0)
    wait_scatter(0)
    plsc.subcore_barrier()
    pltpu.sync_copy(acc.at[pl.ds(r0, _RPT)], out_hbm.at[c, pl.ds(r0, _RPT)])


def _sc_conv(x, e, src, dst, zeros):
    f = pl.kernel(
        _sc_conv_body,
        out_type=jax.ShapeDtypeStruct((_NC, _NP, H), _F32),
        mesh=plsc.VectorSubcoreMesh(core_axis_name="c", subcore_axis_name="s"),
        scratch_types=[
            pltpu.VMEM((2, _CH), jnp.int32),
            pltpu.VMEM((2, _CH), jnp.int32),
            pltpu.VMEM((2, _CH, H), _F32),
            pltpu.VMEM((2, _CH, H), _F32),
            pltpu.VMEM_SHARED((_NP, H), _F32),
            pltpu.SemaphoreType.DMA((2,)),
            pltpu.SemaphoreType.DMA((2,)),
            pltpu.SemaphoreType.DMA((2,)),
        ],
    )
    return f(x, e, src, dst, zeros)


def _mlp_stats_body(scale_ref, parts_ref, x_ref, w1_ref, b1_ref, w2_ref,
                    b2_ref, n2g_ref, h_ref, stats_ref):
    i = pl.program_id(0)
    agg = parts_ref[0] + parts_ref[1]
    h = agg + scale_ref[0, 0] * x_ref[...]
    h = jax.nn.silu(_dot(h, w1_ref[...]) + b1_ref[...])
    h = jax.nn.silu(_dot(h, w2_ref[...]) + b2_ref[...])
    h_ref[...] = h
    oh = _onehot(n2g_ref[0, 0, :])
    dn = (((0,), (0,)), ((), ()))
    s1 = jnp.sum(lax.dot_general(oh, h, dn, preferred_element_type=_F32), 1)
    s2 = jnp.sum(lax.dot_general(oh, h * h, dn, preferred_element_type=_F32), 1)
    s0 = jnp.sum(oh, 0)
    blk = jnp.concatenate(
        [s0[None], s1[None], s2[None], jnp.zeros((5, G), _F32)], 0)

    @pl.when(i == 0)
    def _():
        stats_ref[...] = jnp.zeros_like(stats_ref)

    stats_ref[...] += blk


def _node_mlp_stats(scale, parts, x, w1, b1, w2, b2, n2g3):
    return pl.pallas_call(
        _mlp_stats_body,
        grid=(NB,),
        in_specs=[pl.BlockSpec((1, 1), lambda i: (0, 0)),
                  pl.BlockSpec((_NC, BN, H), lambda i: (0, i, 0)),
                  pl.BlockSpec((BN, H), lambda i: (i, 0)),
                  pl.BlockSpec((H, H), lambda i: (0, 0)),
                  pl.BlockSpec((1, H), lambda i: (0, 0)),
                  pl.BlockSpec((H, H), lambda i: (0, 0)),
                  pl.BlockSpec((1, H), lambda i: (0, 0)),
                  pl.BlockSpec((1, 1, BN), lambda i: (i, 0, 0))],
        out_specs=[pl.BlockSpec((BN, H), lambda i: (i, 0)),
                   pl.BlockSpec((8, G), lambda i: (0, 0))],
        out_shape=[jax.ShapeDtypeStruct((N, H), _F32),
                   jax.ShapeDtypeStruct((8, G), _F32)],
    )(scale, parts, x, w1, b1, w2, b2, n2g3)


def _norm_body(h_ref, x_ref, stats_ref, n2g_ref, w_ref, b_ref, o_ref):
    s0 = stats_ref[0, :]
    s1 = stats_ref[1, :]
    s2 = stats_ref[2, :]
    norm = jnp.maximum(s0, 1.0) * jnp.float32(H)
    mean = s1 / norm
    k = s0 * jnp.float32(H)
    var = (s2 - 2.0 * mean * s1 + mean * mean * k) / norm
    inv = lax.rsqrt(var + 1e-5)
    oh = _onehot(n2g_ref[0, 0, :])
    mean_n = jnp.sum(oh * mean[None, :], 1)
    inv_n = jnp.sum(oh * inv[None, :], 1)
    gn = (h_ref[...] - mean_n[:, None]) * inv_n[:, None] * w_ref[...] + b_ref[...]
    o_ref[...] = jnp.maximum((gn + x_ref[...]) * 0.5, 0.0)


def _apply_norm(h, x, stats, n2g3, w, b):
    return pl.pallas_call(
        _norm_body,
        grid=(NB,),
        in_specs=[pl.BlockSpec((BN, H), lambda i: (i, 0)),
                  pl.BlockSpec((BN, H), lambda i: (i, 0)),
                  pl.BlockSpec((8, G), lambda i: (0, 0)),
                  pl.BlockSpec((1, 1, BN), lambda i: (i, 0, 0)),
                  pl.BlockSpec((1, H), lambda i: (0, 0)),
                  pl.BlockSpec((1, H), lambda i: (0, 0))],
        out_specs=pl.BlockSpec((BN, H), lambda i: (i, 0)),
        out_shape=jax.ShapeDtypeStruct((N, H), _F32),
    )(h, x, stats, n2g3, w, b)


def _final_body(x2_ref, xin_ref, wf_ref, bf_ref, w1_ref, b1_ref, w2_ref,
                b2_ref, n2g_ref, xf_ref, z1_ref):
    i = pl.program_id(0)
    xf = (_dot(x2_ref[...], wf_ref[0:H, :])
          + _dot(xin_ref[...], wf_ref[H:H + FV, :]) + bf_ref[...])
    xf = jax.nn.silu(xf)
    xf_ref[...] = xf
    g1 = _dot(xf, w1_ref[...]) + b1_ref[...]
    g2 = _dot(xf, w2_ref[...]) + b2_ref[...]
    hh = g1 * jax.nn.sigmoid(g2)
    oh = _onehot(n2g_ref[0, 0, :])
    dn = (((0,), (0,)), ((), ()))
    blk = lax.dot_general(oh, hh, dn, preferred_element_type=_F32)

    @pl.when(i == 0)
    def _():
        z1_ref[...] = jnp.zeros_like(z1_ref)

    z1_ref[...] += blk


def _final_readout(x2, x_inp, wf, bf, w1, b1, w2, b2, n2g3):
    return pl.pallas_call(
        _final_body,
        grid=(NB,),
        in_specs=[pl.BlockSpec((BN, H), lambda i: (i, 0)),
                  pl.BlockSpec((BN, FV), lambda i: (i, 0)),
                  pl.BlockSpec((H + FV, H), lambda i: (0, 0)),
                  pl.BlockSpec((1, H), lambda i: (0, 0)),
                  pl.BlockSpec((H, GV), lambda i: (0, 0)),
                  pl.BlockSpec((1, GV), lambda i: (0, 0)),
                  pl.BlockSpec((H, GV), lambda i: (0, 0)),
                  pl.BlockSpec((1, GV), lambda i: (0, 0)),
                  pl.BlockSpec((1, 1, BN), lambda i: (i, 0, 0))],
        out_specs=[pl.BlockSpec((BN, H), lambda i: (i, 0)),
                   pl.BlockSpec((G, GV), lambda i: (0, 0))],
        out_shape=[jax.ShapeDtypeStruct((N, H), _F32),
                   jax.ShapeDtypeStruct((G, GV), _F32)],
    )(x2, x_inp, wf, bf, w1, b1, w2, b2, n2g3)


def _readout_mix_body(z1_ref, stats_ref, w3_ref, b3_ref, z_ref):
    c = jnp.maximum(stats_ref[0, :], 1.0)
    z1 = z1_ref[...]
    z2 = z1 / c[:, None]
    z = (_dot(z1, w3_ref[0:GV, :]) + _dot(z2, w3_ref[GV:2 * GV, :])
         + b3_ref[...])
    z_ref[...] = jax.nn.silu(z)


def _readout_mix(z1, stats, w3, b3):
    return pl.pallas_call(
        _readout_mix_body,
        in_specs=[pl.BlockSpec((G, GV), lambda: (0, 0)),
                  pl.BlockSpec((8, G), lambda: (0, 0)),
                  pl.BlockSpec((2 * GV, GV), lambda: (0, 0)),
                  pl.BlockSpec((1, GV), lambda: (0, 0))],
        out_specs=pl.BlockSpec((G, GV), lambda: (0, 0)),
        out_shape=jax.ShapeDtypeStruct((G, GV), _F32),
    )(z1, stats, w3, b3)


def kernel(x_inp, edge_index, edge_attr, node2graph, params):
    src = edge_index[0]
    dst = edge_index[1]
    n2g3 = node2graph.reshape(NB, 1, BN)
    zeros = jnp.zeros((_NP, H), _F32)

    p = params
    row = lambda v: v.reshape(1, -1)

    x = _node_emb(x_inp, p['node_emb']['W'], row(p['node_emb']['b']))

    b0c = p['blocks'][0]['conv2']
    b1c = p['blocks'][1]['conv2']
    wcat = jnp.concatenate([b0c['We'], b1c['We']],
                           axis=1).astype(jnp.bfloat16)
    bcat = jnp.concatenate([b0c['be'], b1c['be']]).reshape(1, -1)
    e0, e1 = _edge_mlp(edge_attr, p['edge_emb']['W'], row(p['edge_emb']['b']),
                       wcat, bcat)

    stats = None
    for bp, e in ((p['blocks'][0], e0), (p['blocks'][1], e1)):
        cp = bp['conv2']
        parts = _sc_conv(x, e, src, dst, zeros)
        scale = (1.0 + cp['eps']).reshape(1, 1)
        h, stats = _node_mlp_stats(scale, parts, x, cp['W1'], row(cp['b1']),
                                   cp['W2'], row(cp['b2']), n2g3)
        x = _apply_norm(h, x, stats, n2g3, row(bp['norm2']['w']),
                        row(bp['norm2']['b']))

    r = p['readout']
    xf, z1 = _final_readout(x, x_inp, p['final']['W'], row(p['final']['b']),
                            r['W1'], row(r['b1']), r['W2'], row(r['b2']),
                            n2g3)
    z = _readout_mix(z1, stats, r['W3'], row(r['b3']))
    return (xf, z)
